# initial kernel scaffold (unmeasured)
import jax
import jax.numpy as jnp
from jax import lax
from jax.experimental import pallas as pl
from jax.experimental.pallas import tpu as pltpu

N_DEV = 4
B, SQ, D_MODEL = 2, 512, 768
SKV_LOC = 512
H_LOC, DH = 8, 64
BLK = 64
N_QB = SQ // BLK
BF16 = jnp.bfloat16


def kernel(x, Wq, K_ext, V_ext, Wo):
    def body(x_ref, wq_ref, k_ref, v_ref, wo_ref, out_ref,
             kv_full, send_buf, partials,
             a2a_send_sems, a2a_recv_sems, p2_send_sems, p2_recv_sems):
        my = lax.axis_index("i")

        barrier = pltpu.get_barrier_semaphore()
        for off in (1, 2, 3):
            pl.semaphore_signal(
                barrier, inc=1,
                device_id=((my + off) % N_DEV,),
                device_id_type=pl.DeviceIdType.MESH,
            )
        pl.semaphore_wait(barrier, 3)

        kv_full[my, 0] = k_ref[:, :, pl.ds(my * H_LOC, H_LOC), :].astype(BF16)
        kv_full[my, 1] = v_ref[:, :, pl.ds(my * H_LOC, H_LOC), :].astype(BF16)

        sends = []
        for off in (1, 2, 3):
            d = (my + off) % N_DEV
            send_buf[off - 1, 0] = k_ref[:, :, pl.ds(d * H_LOC, H_LOC), :].astype(BF16)
            send_buf[off - 1, 1] = v_ref[:, :, pl.ds(d * H_LOC, H_LOC), :].astype(BF16)
            rdma = pltpu.make_async_remote_copy(
                src_ref=send_buf.at[off - 1],
                dst_ref=kv_full.at[my],
                send_sem=a2a_send_sems.at[off - 1],
                recv_sem=a2a_recv_sems.at[my],
                device_id=(d,),
                device_id_type=pl.DeviceIdType.MESH,
            )
            rdma.start()
            sends.append(rdma)

        xq = x_ref[...].reshape(B * SQ, D_MODEL).astype(BF16)
        wq = wq_ref[...].astype(BF16)
        q = lax.dot_general(xq, wq, (((1,), (0,)), ((), ())),
                            preferred_element_type=jnp.float32)
        q = (q * 0.125).astype(BF16).reshape(B, SQ, H_LOC, DH)

        for off in (1, 2, 3):
            j = (my + off) % N_DEV
            recv = pltpu.make_async_remote_copy(
                src_ref=send_buf.at[0],
                dst_ref=kv_full.at[j],
                send_sem=a2a_send_sems.at[0],
                recv_sem=a2a_recv_sems.at[j],
                device_id=(j,),
                device_id_type=pl.DeviceIdType.MESH,
            )
            recv.wait_recv()
        for r in sends:
            r.wait_send()

        ctx_blocks = [None] * N_QB
        for c in range(4):
            q_c = jnp.concatenate(
                [q[:, BLK * c:BLK * (c + 1)],
                 q[:, BLK * (c + 4):BLK * (c + 5)]], axis=1)
            k_slabs, v_slabs = [], []
            for t in range(8):
                j0 = BLK * c + 4 * BLK * t
                s, loc = divmod(j0, SKV_LOC)
                k_slabs.append(kv_full[s, 0, :, loc:loc + BLK])
                v_slabs.append(kv_full[s, 1, :, loc:loc + BLK])
            k_c = jnp.concatenate(k_slabs, axis=1)
            v_c = jnp.concatenate(v_slabs, axis=1)
            scores = jnp.einsum('bqhd,bkhd->bhqk', q_c, k_c,
                                preferred_element_type=jnp.float32)
            m = jnp.max(scores, axis=-1, keepdims=True)
            w = jnp.exp(scores - m)
            w = w / jnp.sum(w, axis=-1, keepdims=True)
            ctx_c = jnp.einsum('bhqk,bkhd->bqhd', w.astype(BF16), v_c,
                               preferred_element_type=jnp.float32)
            ctx_blocks[c] = ctx_c[:, :BLK]
            ctx_blocks[c + 4] = ctx_c[:, BLK:]
        ctx = jnp.concatenate(ctx_blocks, axis=1)
        ctx = ctx.astype(BF16).reshape(B * SQ, H_LOC * DH)

        wo = wo_ref[...].astype(BF16)
        part = lax.dot_general(ctx, wo, (((1,), (0,)), ((), ())),
                               preferred_element_type=jnp.float32)
        partials[my] = part.reshape(B, SQ, D_MODEL)

        p2 = []
        for off in (1, 2, 3):
            d = (my + off) % N_DEV
            rdma = pltpu.make_async_remote_copy(
                src_ref=partials.at[my],
                dst_ref=partials.at[my],
                send_sem=p2_send_sems.at[off - 1],
                recv_sem=p2_recv_sems.at[my],
                device_id=(d,),
                device_id_type=pl.DeviceIdType.MESH,
            )
            rdma.start()
            p2.append(rdma)
        for off in (1, 2, 3):
            j = (my + off) % N_DEV
            recv = pltpu.make_async_remote_copy(
                src_ref=partials.at[my],
                dst_ref=partials.at[j],
                send_sem=p2_send_sems.at[0],
                recv_sem=p2_recv_sems.at[j],
                device_id=(j,),
                device_id_type=pl.DeviceIdType.MESH,
            )
            recv.wait_recv()
        for r in p2:
            r.wait_send()

        out_ref[...] = (partials[0] + partials[1]) + (partials[2] + partials[3])

    return pl.pallas_call(
        body,
        out_shape=jax.ShapeDtypeStruct((B, SQ, D_MODEL), jnp.float32),
        in_specs=[pl.BlockSpec(memory_space=pltpu.VMEM)] * 5,
        out_specs=pl.BlockSpec(memory_space=pltpu.VMEM),
        scratch_shapes=[
            pltpu.VMEM((N_DEV, 2, B, SKV_LOC, H_LOC, DH), BF16),
            pltpu.VMEM((N_DEV - 1, 2, B, SKV_LOC, H_LOC, DH), BF16),
            pltpu.VMEM((N_DEV, B, SQ, D_MODEL), jnp.float32),
            pltpu.SemaphoreType.DMA((N_DEV - 1,)),
            pltpu.SemaphoreType.DMA((N_DEV,)),
            pltpu.SemaphoreType.DMA((N_DEV - 1,)),
            pltpu.SemaphoreType.DMA((N_DEV,)),
        ],
        compiler_params=pltpu.CompilerParams(collective_id=0),
    )(x, Wq, K_ext, V_ext, Wo)


# baseline (device time: 134654 ns/iter reference)
import jax
import jax.numpy as jnp
from jax import lax
from jax.experimental import pallas as pl
from jax.experimental.pallas import tpu as pltpu

N_DEV = 4
B, SQ, D_MODEL = 2, 512, 768
SKV_LOC = 512
H_LOC, DH = 8, 64
HD = H_LOC * DH
BLK = 64
N_QB = SQ // BLK
BF16 = jnp.bfloat16


def kernel(x, Wq, K_ext, V_ext, Wo):
    k2 = K_ext.reshape(B, SKV_LOC, N_DEV * HD)
    v2 = V_ext.reshape(B, SKV_LOC, N_DEV * HD)

    def body(x_ref, wq_ref, k_ref, v_ref, wo_ref, out_ref,
             kv_full, send_buf, stage, partials,
             stage_sems, a2a_send_sems, a2a_recv_sems,
             p2_send_sems, p2_recv_sems):
        my = lax.axis_index("i")

        barrier = pltpu.get_barrier_semaphore()
        for off in (1, 2, 3):
            pl.semaphore_signal(
                barrier, inc=1,
                device_id=((my + off) % N_DEV,),
                device_id_type=pl.DeviceIdType.MESH,
            )
        pl.semaphore_wait(barrier, 3)

        def stage_slice(src_ref, d):
            cp = pltpu.make_async_copy(
                src_ref.at[:, :, pl.ds(d * HD, HD)],
                stage,
                stage_sems.at[0],
            )
            cp.start()
            cp.wait()
            return stage[...].astype(BF16)

        kv_full[my, 0] = stage_slice(k_ref, my)
        kv_full[my, 1] = stage_slice(v_ref, my)

        sends = []
        for off in (1, 2, 3):
            d = (my + off) % N_DEV
            send_buf[off - 1, 0] = stage_slice(k_ref, d)
            send_buf[off - 1, 1] = stage_slice(v_ref, d)
            rdma = pltpu.make_async_remote_copy(
                src_ref=send_buf.at[off - 1],
                dst_ref=kv_full.at[my],
                send_sem=a2a_send_sems.at[off - 1],
                recv_sem=a2a_recv_sems.at[my],
                device_id=(d,),
                device_id_type=pl.DeviceIdType.MESH,
            )
            rdma.start()
            sends.append(rdma)

        xq = x_ref[...].reshape(B * SQ, D_MODEL).astype(BF16)
        wq = wq_ref[...].astype(BF16)
        q = lax.dot_general(xq, wq, (((1,), (0,)), ((), ())),
                            preferred_element_type=jnp.float32)
        q = (q * 0.125).astype(BF16)

        for off in (1, 2, 3):
            j = (my + off) % N_DEV
            recv = pltpu.make_async_remote_copy(
                src_ref=send_buf.at[0],
                dst_ref=kv_full.at[j],
                send_sem=a2a_send_sems.at[0],
                recv_sem=a2a_recv_sems.at[j],
                device_id=(j,),
                device_id_type=pl.DeviceIdType.MESH,
            )
            recv.wait_recv()
        for r in sends:
            r.wait_send()

        ctx_blocks = [[None] * N_QB for _ in range(B)]
        for c in range(4):
            k_slabs, v_slabs = [], []
            for t in range(8):
                j0 = BLK * c + 4 * BLK * t
                s, loc = divmod(j0, SKV_LOC)
                k_slabs.append(kv_full[s, 0, :, loc:loc + BLK])
                v_slabs.append(kv_full[s, 1, :, loc:loc + BLK])
            k_c = jnp.concatenate(k_slabs, axis=1)
            v_c = jnp.concatenate(v_slabs, axis=1)
            for b in range(B):
                q_cb = jnp.concatenate(
                    [q[b * SQ + BLK * c:b * SQ + BLK * (c + 1)],
                     q[b * SQ + BLK * (c + 4):b * SQ + BLK * (c + 5)]],
                    axis=0)
                ctx_h = []
                for h in range(H_LOC):
                    q_h = q_cb[:, DH * h:DH * (h + 1)]
                    k_h = k_c[b, :, DH * h:DH * (h + 1)]
                    v_h = v_c[b, :, DH * h:DH * (h + 1)]
                    scores = lax.dot_general(
                        q_h, k_h, (((1,), (1,)), ((), ())),
                        preferred_element_type=jnp.float32)
                    m = jnp.max(scores, axis=-1, keepdims=True)
                    w = jnp.exp(scores - m)
                    w = w / jnp.sum(w, axis=-1, keepdims=True)
                    ctx_h.append(lax.dot_general(
                        w.astype(BF16), v_h, (((1,), (0,)), ((), ())),
                        preferred_element_type=jnp.float32))
                ctx_cb = jnp.concatenate(ctx_h, axis=1)
                ctx_blocks[b][c] = ctx_cb[:BLK]
                ctx_blocks[b][c + 4] = ctx_cb[BLK:]
        ctx = jnp.concatenate(
            [blk for b in range(B) for blk in ctx_blocks[b]],
            axis=0).astype(BF16)

        wo = wo_ref[...].astype(BF16)
        part = lax.dot_general(ctx, wo, (((1,), (0,)), ((), ())),
                               preferred_element_type=jnp.float32)
        partials[my] = part.astype(BF16).reshape(B, SQ, D_MODEL)

        p2 = []
        for off in (1, 2, 3):
            d = (my + off) % N_DEV
            rdma = pltpu.make_async_remote_copy(
                src_ref=partials.at[my],
                dst_ref=partials.at[my],
                send_sem=p2_send_sems.at[off - 1],
                recv_sem=p2_recv_sems.at[my],
                device_id=(d,),
                device_id_type=pl.DeviceIdType.MESH,
            )
            rdma.start()
            p2.append(rdma)
        for off in (1, 2, 3):
            j = (my + off) % N_DEV
            recv = pltpu.make_async_remote_copy(
                src_ref=partials.at[my],
                dst_ref=partials.at[j],
                send_sem=p2_send_sems.at[0],
                recv_sem=p2_recv_sems.at[j],
                device_id=(j,),
                device_id_type=pl.DeviceIdType.MESH,
            )
            recv.wait_recv()
        for r in p2:
            r.wait_send()

        acc = partials[0].astype(jnp.float32) + partials[1].astype(jnp.float32)
        acc = acc + partials[2].astype(jnp.float32)
        out_ref[...] = acc + partials[3].astype(jnp.float32)

    return pl.pallas_call(
        body,
        out_shape=jax.ShapeDtypeStruct((B, SQ, D_MODEL), jnp.float32),
        in_specs=[
            pl.BlockSpec(memory_space=pltpu.VMEM),
            pl.BlockSpec(memory_space=pltpu.VMEM),
            pl.BlockSpec(memory_space=pltpu.MemorySpace.HBM),
            pl.BlockSpec(memory_space=pltpu.MemorySpace.HBM),
            pl.BlockSpec(memory_space=pltpu.VMEM),
        ],
        out_specs=pl.BlockSpec(memory_space=pltpu.VMEM),
        scratch_shapes=[
            pltpu.VMEM((N_DEV, 2, B, SKV_LOC, HD), BF16),
            pltpu.VMEM((N_DEV - 1, 2, B, SKV_LOC, HD), BF16),
            pltpu.VMEM((B, SKV_LOC, HD), jnp.float32),
            pltpu.VMEM((N_DEV, B, SQ, D_MODEL), BF16),
            pltpu.SemaphoreType.DMA((1,)),
            pltpu.SemaphoreType.DMA((N_DEV - 1,)),
            pltpu.SemaphoreType.DMA((N_DEV,)),
            pltpu.SemaphoreType.DMA((N_DEV - 1,)),
            pltpu.SemaphoreType.DMA((N_DEV,)),
        ],
        compiler_params=pltpu.CompilerParams(collective_id=0),
    )(x, Wq, k2, v2, Wo)
